# baseline (speedup 1.0000x reference)
import functools

import jax
import jax.numpy as jnp
from jax import lax
from jax.experimental import pallas as pl
from jax.experimental.pallas import tpu as pltpu
from jax.experimental.pallas import tpu_sc as plsc

DIM = 64
LANES = 16
QUARTERS = DIM // LANES
NUM_CORES = 2
NUM_SUBCORES = 16
NW = NUM_CORES * NUM_SUBCORES
CHUNK = 128
MARGIN = 1.0

_GATHER_DNUMS = lax.GatherDimensionNumbers(
    offset_dims=(), collapsed_slice_dims=(0,), start_index_map=(0,))


def _lane_shuffle(x, perm):
    return lax.gather(
        x, perm[:, None], _GATHER_DNUMS, slice_sizes=(1,),
        mode=lax.GatherScatterMode.PROMISE_IN_BOUNDS)


def _make_transe(B):
    assert B % NW == 0
    per_w = B // NW
    assert per_w % CHUNK == 0
    nch = per_w // CHUNK
    mesh = plsc.VectorSubcoreMesh(core_axis_name="c", subcore_axis_name="s")
    buf_shape = (CHUNK, 2 * DIM)

    @functools.partial(
        pl.kernel,
        out_type=jax.ShapeDtypeStruct((NW, LANES), jnp.float32),
        mesh=mesh,
        scratch_types=[
            pltpu.VMEM((6, nch, CHUNK), jnp.int32),
            pltpu.VMEM((6, nch, CHUNK), jnp.int32),
            pltpu.VMEM(buf_shape, jnp.float32),
            pltpu.VMEM(buf_shape, jnp.float32),
            pltpu.VMEM(buf_shape, jnp.float32),
            pltpu.VMEM(buf_shape, jnp.float32),
            pltpu.VMEM(buf_shape, jnp.float32),
            pltpu.VMEM(buf_shape, jnp.float32),
            pltpu.VMEM((LANES,), jnp.float32),
            pltpu.SemaphoreType.DMA,
        ],
    )
    def transe_kernel(pair_hbm, half_hbm, etab, rtab, out_hbm, pair_v,
                      half_v, bph, bpr, bpt, bnh, bnr, bnt, ovec, sem):
        wid = lax.axis_index("s") * NUM_CORES + lax.axis_index("c")
        pltpu.sync_copy(pair_hbm.at[wid], pair_v)
        pltpu.sync_copy(half_hbm.at[wid], half_v)
        bufs = (bph, bpr, bpt, bnh, bnr, bnt)
        tabs = (etab, rtab, etab, etab, rtab, etab)

        def chunk_body(c, loss_vec):
            copies = [
                pltpu.async_copy(tabs[j].at[pair_v.at[j, c]], bufs[j], sem)
                for j in range(6)
            ]
            for cp in copies:
                cp.wait()

            def group_body(g, lv):
                base = g * LANES
                half_vecs = [half_v[j, c, pl.ds(base, LANES)]
                             for j in range(6)]
                for k in range(LANES):
                    i = base + k
                    offs = [half_vecs[j][k] * DIM for j in range(6)]
                    gap = None
                    for q in range(QUARTERS):
                        qo = q * LANES
                        p = jnp.abs(
                            bph[i, pl.ds(offs[0] + qo, LANES)]
                            + bpr[i, pl.ds(offs[1] + qo, LANES)]
                            - bpt[i, pl.ds(offs[2] + qo, LANES)])
                        n = jnp.abs(
                            bnh[i, pl.ds(offs[3] + qo, LANES)]
                            + bnr[i, pl.ds(offs[4] + qo, LANES)]
                            - bnt[i, pl.ds(offs[5] + qo, LANES)])
                        gap = p - n if gap is None else gap + (p - n)
                    s = gap
                    for b in (8, 4, 2, 1):
                        perm = lax.iota(jnp.int32, LANES) ^ b
                        s = s + _lane_shuffle(s, perm)
                    hinge = jnp.maximum(MARGIN + s, 0.0)
                    lane_hit = lax.iota(jnp.int32, LANES) == k
                    lv = lv + jnp.where(lane_hit, hinge, 0.0)
                return lv

            return lax.fori_loop(0, CHUNK // LANES, group_body, loss_vec)

        loss_vec = lax.fori_loop(0, nch, chunk_body,
                                 jnp.zeros((LANES,), jnp.float32))
        ovec[...] = loss_vec
        pltpu.sync_copy(ovec, out_hbm.at[wid])

    return transe_kernel


def kernel(positive_triples, negative_triples, entity_embeddings,
           relation_embeddings):
    B = positive_triples.shape[0]
    per_w = B // NW
    nch = per_w // CHUNK
    idx = jnp.stack(
        [
            positive_triples[:, 0],
            positive_triples[:, 1],
            positive_triples[:, 2],
            negative_triples[:, 0],
            negative_triples[:, 1],
            negative_triples[:, 2],
        ],
        axis=0,
    )
    idx = idx.reshape(6, NW, nch, CHUNK).transpose(1, 0, 2, 3)
    pair = idx >> 1
    half = idx & 1
    etab2 = entity_embeddings.reshape(-1, 2 * DIM)
    rtab2 = relation_embeddings.reshape(-1, 2 * DIM)
    partials = _make_transe(B)(pair, half, etab2, rtab2)
    return jnp.sum(partials) * (1.0 / B)
